# baseline (speedup 1.0000x reference)
import functools

import jax
import jax.numpy as jnp
from jax import lax
from jax.experimental import pallas as pl
from jax.experimental.pallas import tpu as pltpu
from jax.experimental.pallas import tpu_sc as plsc

HIDDEN = 768
D_FF = 2048
EPS = 1e-06
INTERPRET = False


def _rms(x, w):
    v = jnp.mean(x * x, axis=-1, keepdims=True)
    return x * jax.lax.rsqrt(v + EPS) * w


def _k1_body(x_ref, decnw_ref, wg_ref, wu_ref, wd_ref, pnw_ref, wmu_ref,
             wlv_ref, wcr_ref, bcr_ref, proc_ref, g_ref, logit_ref, s1_ref):
    x = x_ref[...]
    h = _rms(x, decnw_ref[...]).astype(jnp.bfloat16)
    hg = jnp.dot(h, wg_ref[...], preferred_element_type=jnp.float32)
    hu = jnp.dot(h, wu_ref[...], preferred_element_type=jnp.float32)
    act = ((hg * jax.nn.sigmoid(hg)) * hu).astype(jnp.bfloat16)
    delta = jnp.dot(act, wd_ref[...], preferred_element_type=jnp.float32)
    proc_ref[...] = x + delta

    xn = _rms(x, pnw_ref[...]).astype(jnp.bfloat16)
    mu = jnp.dot(xn, wmu_ref[...], preferred_element_type=jnp.float32)
    lv = jnp.dot(xn, wlv_ref[...], preferred_element_type=jnp.float32)
    d_st = jnp.sum(delta * delta, axis=-1) / float(HIDDEN)
    d_ch = 0.5 * jnp.mean(
        lv + (1.0 + (delta - mu) ** 2) * jnp.exp(-lv) - 1.0, axis=-1)
    g_ref[0, 0, :] = jax.nn.sigmoid(d_st - d_ch)

    logit = jnp.dot(x, wcr_ref[...],
                    preferred_element_type=jnp.float32)[:, 0] + bcr_ref[0, :]
    logit_ref[0, 0, :] = logit
    spl = jnp.maximum(logit, 0.0) + jnp.log1p(jnp.exp(-jnp.abs(logit)))
    s1_ref[0, 0, :] = spl


def _k2_body(g_ref, logit_ref, s1_ref, idx_ref, gsel_ref, loss_ref, *, k,
             n_tok):
    B, T = g_ref.shape
    g = g_ref[...]
    gi = jax.lax.bitcast_convert_type(g, jnp.int32)

    def bs_step(i, lo):
        cand = lo | jnp.left_shift(jnp.int32(1), 30 - i)
        cnt = jnp.sum((gi >= cand).astype(jnp.int32), axis=1, keepdims=True)
        return jnp.where(cnt >= k, cand, lo)

    lo = jnp.zeros((B, 1), jnp.int32)
    thr = jax.lax.fori_loop(0, 31, bs_step, lo)

    m_gt = gi > thr
    n_gt = jnp.sum(m_gt.astype(jnp.int32), axis=1, keepdims=True)
    r = k - n_gt
    m_eq = gi == thr
    c = m_eq.astype(jnp.int32)
    s = 1
    while s < T:
        c = c + jnp.concatenate(
            [jnp.zeros((B, s), jnp.int32), c[:, :-s]], axis=1)
        s *= 2
    mask = m_gt | (m_eq & (c <= r))

    lsel = jnp.sum(jnp.where(mask, logit_ref[...], 0.0))
    loss = (jnp.sum(s1_ref[...]) - lsel) / float(n_tok)
    loss_ref[0, :] = jnp.full((128,), loss)

    mi = mask.astype(jnp.int32)
    pc = mi
    s = 1
    while s < T:
        pc = pc + jnp.concatenate(
            [jnp.zeros((B, s), jnp.int32), pc[:, :-s]], axis=1)
        s *= 2

    PK = 128
    tio = lax.broadcasted_iota(jnp.int32, (1, T), 1)
    rows_idx = []
    rows_g = []
    for b in range(B):
        pcb = pc[b:b + 1, :]
        mb = mask[b:b + 1, :]
        gb = g[b:b + 1, :]
        chunks_i = []
        chunks_g = []
        for c0 in range(0, k, PK):
            prow = lax.broadcasted_iota(jnp.int32, (PK, 1), 0) + (c0 + 1)
            eqm = (pcb == prow) & mb
            chunks_i.append(
                jnp.sum(jnp.where(eqm, tio + b * T, 0), axis=1))
            chunks_g.append(jnp.sum(jnp.where(eqm, gb, 0.0), axis=1))
        rows_idx.append(jnp.concatenate(chunks_i))
        rows_g.append(jnp.concatenate(chunks_g))
    idx_ref[...] = jnp.stack(rows_idx)
    gsel_ref[...] = jnp.stack(rows_g)


def _gather_body(proc_hbm, idx_hbm, sel_hbm, idx_v, rows_v, sem, *, K):
    c = lax.axis_index("c")
    s = lax.axis_index("s")
    bpw = K // 16
    base = c * K + s * bpw
    pltpu.sync_copy(idx_hbm.at[pl.ds(base, bpw)], idx_v)
    pltpu.async_copy(proc_hbm.at[idx_v], rows_v, sem).wait()
    pltpu.sync_copy(rows_v, sel_hbm.at[pl.ds(base, bpw)])


def _scatter_body(proc_hbm, gated_hbm, idx_hbm, final_hbm, buf_v, idx_v,
                  rows_v, sem, *, T, K):
    c = lax.axis_index("c")
    s = lax.axis_index("s")
    rows_per_tile = T // 16
    chunk = 32
    row0 = c * T + s * rows_per_tile

    def cp(u, carry):
        src = proc_hbm.at[pl.ds(row0 + u * chunk, chunk)]
        dst = final_hbm.at[pl.ds(row0 + u * chunk, chunk)]
        pltpu.sync_copy(src, buf_v)
        pltpu.sync_copy(buf_v, dst)
        return carry

    lax.fori_loop(0, rows_per_tile // chunk, cp, jnp.int32(0))
    plsc.subcore_barrier()

    bpw = K // 16
    base = c * K + s * bpw
    pltpu.sync_copy(idx_hbm.at[pl.ds(base, bpw)], idx_v)
    pltpu.sync_copy(gated_hbm.at[pl.ds(base, bpw)], rows_v)
    pltpu.async_copy(rows_v, final_hbm.at[idx_v], sem).wait()


def _k4_body(sel_ref, gsel_ref, nw_ref, wg_ref, wu_ref, wd_ref, out_ref):
    p = sel_ref[...]
    h = _rms(p, nw_ref[...]).astype(jnp.bfloat16)
    hg = jnp.dot(h, wg_ref[...], preferred_element_type=jnp.float32)
    hu = jnp.dot(h, wu_ref[...], preferred_element_type=jnp.float32)
    act = ((hg * jax.nn.sigmoid(hg)) * hu).astype(jnp.bfloat16)
    delta = jnp.dot(act, wd_ref[...], preferred_element_type=jnp.float32)
    out_ref[...] = p + gsel_ref[...].reshape(-1, 1) * delta


def kernel(hidden_states, prior_norm_w, W_mu, W_logvar, dec_norm_w, dec_Wg,
           dec_Wu, dec_Wd, dyn_norm_w, dyn_Wg, dyn_Wu, dyn_Wd, w_cr, b_cr):
    B, T, D = hidden_states.shape
    N = B * T
    BLK = 256
    nblk = N // BLK
    k = max(1, int(T * 0.25))

    x2d = hidden_states.reshape(N, D)
    row = lambda w: w.reshape(1, D)
    wcol = w_cr.reshape(D, 1)
    bf = lambda w: w.astype(jnp.bfloat16)
    bcr = jnp.full((1, BLK), b_cr, jnp.float32)

    full = lambda shape: pl.BlockSpec(shape, lambda *_: (0,) * len(shape))
    tokb = pl.BlockSpec((BLK, D), lambda i: (i, 0))

    proc, g2, logit2, s12 = pl.pallas_call(
        _k1_body,
        grid=(nblk,),
        in_specs=[
            tokb, full((1, D)), full((D, D_FF)), full((D, D_FF)),
            full((D_FF, D)), full((1, D)), full((D, D)), full((D, D)),
            full((D, 1)), full((1, BLK)),
        ],
        out_specs=[
            tokb,
            pl.BlockSpec((1, 1, BLK), lambda i: (i, 0, 0)),
            pl.BlockSpec((1, 1, BLK), lambda i: (i, 0, 0)),
            pl.BlockSpec((1, 1, BLK), lambda i: (i, 0, 0)),
        ],
        out_shape=[
            jax.ShapeDtypeStruct((N, D), jnp.float32),
            jax.ShapeDtypeStruct((nblk, 1, BLK), jnp.float32),
            jax.ShapeDtypeStruct((nblk, 1, BLK), jnp.float32),
            jax.ShapeDtypeStruct((nblk, 1, BLK), jnp.float32),
        ],
        interpret=INTERPRET,
    )(x2d, row(dec_norm_w), bf(dec_Wg), bf(dec_Wu), bf(dec_Wd),
      row(prior_norm_w), bf(W_mu), bf(W_logvar), wcol, bcr)

    g_bt = g2.reshape(B, T)
    logit_bt = logit2.reshape(B, T)

    flat_idx, gsel, loss = pl.pallas_call(
        functools.partial(_k2_body, k=k, n_tok=N),
        in_specs=[full((B, T)), full((B, T)), full((nblk, 1, BLK))],
        out_specs=[full((B, k)), full((B, k)), full((1, 128))],
        out_shape=[
            jax.ShapeDtypeStruct((B, k), jnp.int32),
            jax.ShapeDtypeStruct((B, k), jnp.float32),
            jax.ShapeDtypeStruct((1, 128), jnp.float32),
        ],
        interpret=INTERPRET,
    )(g_bt, logit_bt, s12)

    flat_idx = flat_idx.reshape(B * k)

    mesh = plsc.VectorSubcoreMesh(core_axis_name="c", subcore_axis_name="s")
    gather_fn = functools.partial(
        pl.kernel,
        out_type=jax.ShapeDtypeStruct((B * k, D), jnp.float32),
        mesh=mesh,
        scratch_types=[
            pltpu.VMEM((k // 16,), jnp.int32),
            pltpu.VMEM((k // 16, D), jnp.float32),
            pltpu.SemaphoreType.DMA,
        ],
    )(functools.partial(_gather_body, K=k))
    sel = gather_fn(proc, flat_idx)

    gated = pl.pallas_call(
        _k4_body,
        grid=(B * k // BLK,),
        in_specs=[
            tokb, pl.BlockSpec((1, 1, BLK), lambda i: (i, 0, 0)), full((1, D)),
            full((D, D_FF)), full((D, D_FF)), full((D_FF, D)),
        ],
        out_specs=tokb,
        out_shape=jax.ShapeDtypeStruct((B * k, D), jnp.float32),
        interpret=INTERPRET,
    )(sel, gsel.reshape(B * k // BLK, 1, BLK), row(dyn_norm_w), bf(dyn_Wg),
      bf(dyn_Wu), bf(dyn_Wd))

    scat_fn = functools.partial(
        pl.kernel,
        out_type=jax.ShapeDtypeStruct((N, D), jnp.float32),
        mesh=mesh,
        scratch_types=[
            pltpu.VMEM((32, D), jnp.float32),
            pltpu.VMEM((k // 16,), jnp.int32),
            pltpu.VMEM((k // 16, D), jnp.float32),
            pltpu.SemaphoreType.DMA,
        ],
    )(functools.partial(_scatter_body, T=T, K=k))
    final2d = scat_fn(proc, gated, flat_idx)

    return (final2d.reshape(B, T, D), g_bt, loss[0, 0])


# speedup vs baseline: 1.1495x; 1.1495x over previous
import functools

import jax
import jax.numpy as jnp
from jax import lax
from jax.experimental import pallas as pl
from jax.experimental.pallas import tpu as pltpu
from jax.experimental.pallas import tpu_sc as plsc

HIDDEN = 768
D_FF = 2048
EPS = 1e-06
INTERPRET = False


def _rms(x, w):
    v = jnp.mean(x * x, axis=-1, keepdims=True)
    return x * jax.lax.rsqrt(v + EPS) * w


def _k12_body(x_ref, decnw_ref, wg_ref, wu_ref, wd_ref, pnw_ref, wmu_ref,
              wlv_ref, wcr_ref, bcr_ref, proc_ref, g_ref, idx_ref, gsel_ref,
              loss_ref, g_s, logit_s, spl_s, *, k, n_tok, nblk, tpb):
    i = pl.program_id(0)
    x = x_ref[...]
    h = _rms(x, decnw_ref[...])
    hg = jnp.dot(h, wg_ref[...], preferred_element_type=jnp.float32)
    hu = jnp.dot(h, wu_ref[...], preferred_element_type=jnp.float32)
    act = (hg * jax.nn.sigmoid(hg)) * hu
    delta = jnp.dot(act, wd_ref[...], preferred_element_type=jnp.float32)
    proc_ref[...] = x + delta

    xn = _rms(x, pnw_ref[...])
    mu = jnp.dot(xn, wmu_ref[...], preferred_element_type=jnp.float32)
    lv = jnp.dot(xn, wlv_ref[...], preferred_element_type=jnp.float32)
    d_st = jnp.sum(delta * delta, axis=-1) / float(HIDDEN)
    d_ch = 0.5 * jnp.mean(
        lv + (1.0 + (delta - mu) ** 2) * jnp.exp(-lv) - 1.0, axis=-1)
    gv = jax.nn.sigmoid(d_st - d_ch)
    g_ref[0, 0, :] = gv

    logit = jnp.dot(x, wcr_ref[...],
                    preferred_element_type=jnp.float32)[:, 0] + bcr_ref[0, :]
    spl = jnp.maximum(logit, 0.0) + jnp.log1p(jnp.exp(-jnp.abs(logit)))

    b = i // tpb
    c0 = (i % tpb) * x.shape[0]
    sl = (pl.ds(b, 1), pl.ds(c0, x.shape[0]))
    g_s[sl] = gv.reshape(1, -1)
    logit_s[sl] = logit.reshape(1, -1)
    spl_s[sl] = spl.reshape(1, -1)

    @pl.when(i == nblk - 1)
    def _():
        _k2_work(g_s, logit_s, spl_s, idx_ref, gsel_ref, loss_ref, k=k,
                 n_tok=n_tok)


def _k2_work(g_ref, logit_ref, s1_ref, idx_ref, gsel_ref, loss_ref, *, k,
             n_tok):
    B, T = g_ref.shape
    g = g_ref[...]
    gi = jax.lax.bitcast_convert_type(g, jnp.int32)

    def bs_step(i, lo):
        cand = lo | jnp.left_shift(jnp.int32(1), 30 - i)
        cnt = jnp.sum((gi >= cand).astype(jnp.int32), axis=1, keepdims=True)
        return jnp.where(cnt >= k, cand, lo)

    lo = jnp.zeros((B, 1), jnp.int32)
    thr = jax.lax.fori_loop(0, 31, bs_step, lo)

    m_gt = gi > thr
    n_gt = jnp.sum(m_gt.astype(jnp.int32), axis=1, keepdims=True)
    r = k - n_gt
    m_eq = gi == thr
    c = m_eq.astype(jnp.int32)
    s = 1
    while s < T:
        c = c + jnp.concatenate(
            [jnp.zeros((B, s), jnp.int32), c[:, :-s]], axis=1)
        s *= 2
    mask = m_gt | (m_eq & (c <= r))

    lsel = jnp.sum(jnp.where(mask, logit_ref[...], 0.0))
    loss = (jnp.sum(s1_ref[...]) - lsel) / float(n_tok)
    loss_ref[0, :] = jnp.full((128,), loss)

    mi = mask.astype(jnp.int32)
    pc = mi
    s = 1
    while s < T:
        pc = pc + jnp.concatenate(
            [jnp.zeros((B, s), jnp.int32), pc[:, :-s]], axis=1)
        s *= 2

    PK = 128
    tio = lax.broadcasted_iota(jnp.int32, (1, T), 1)
    rows_idx = []
    rows_g = []
    for b in range(B):
        pcb = pc[b:b + 1, :]
        mb = mask[b:b + 1, :]
        gb = g[b:b + 1, :]
        chunks_i = []
        chunks_g = []
        for c0 in range(0, k, PK):
            prow = lax.broadcasted_iota(jnp.int32, (PK, 1), 0) + (c0 + 1)
            eqm = (pcb == prow) & mb
            chunks_i.append(
                jnp.sum(jnp.where(eqm, tio + b * T, 0), axis=1))
            chunks_g.append(jnp.sum(jnp.where(eqm, gb, 0.0), axis=1))
        rows_idx.append(jnp.concatenate(chunks_i))
        rows_g.append(jnp.concatenate(chunks_g))
    idx_ref[...] = jnp.stack(rows_idx)
    gsel_ref[...] = jnp.stack(rows_g)


def _gather_body(proc_hbm, idx_hbm, sel_hbm, idx_v, rows_v, sem, *, K):
    c = lax.axis_index("c")
    s = lax.axis_index("s")
    bpw = K // 16
    base = c * K + s * bpw
    pltpu.sync_copy(idx_hbm.at[pl.ds(base, bpw)], idx_v)
    pltpu.async_copy(proc_hbm.at[idx_v], rows_v, sem).wait()
    pltpu.sync_copy(rows_v, sel_hbm.at[pl.ds(base, bpw)])


def _scatter_body(proc_hbm, gated_hbm, idx_hbm, final_hbm, buf_v, idx_v,
                  rows_v, sem, *, T, K):
    c = lax.axis_index("c")
    s = lax.axis_index("s")
    rows_per_tile = T // 16
    chunk = 32
    row0 = c * T + s * rows_per_tile

    def cp(u, carry):
        src = proc_hbm.at[pl.ds(row0 + u * chunk, chunk)]
        dst = final_hbm.at[pl.ds(row0 + u * chunk, chunk)]
        pltpu.sync_copy(src, buf_v)
        pltpu.sync_copy(buf_v, dst)
        return carry

    lax.fori_loop(0, rows_per_tile // chunk, cp, jnp.int32(0))
    plsc.subcore_barrier()

    bpw = K // 16
    base = c * K + s * bpw
    pltpu.sync_copy(idx_hbm.at[pl.ds(base, bpw)], idx_v)
    pltpu.sync_copy(gated_hbm.at[pl.ds(base, bpw)], rows_v)
    pltpu.async_copy(rows_v, final_hbm.at[idx_v], sem).wait()


def _k4_body(sel_ref, gsel_ref, nw_ref, wg_ref, wu_ref, wd_ref, out_ref):
    p = sel_ref[...]
    h = _rms(p, nw_ref[...])
    hg = jnp.dot(h, wg_ref[...], preferred_element_type=jnp.float32)
    hu = jnp.dot(h, wu_ref[...], preferred_element_type=jnp.float32)
    act = (hg * jax.nn.sigmoid(hg)) * hu
    delta = jnp.dot(act, wd_ref[...], preferred_element_type=jnp.float32)
    out_ref[...] = p + gsel_ref[...].reshape(-1, 1) * delta


def kernel(hidden_states, prior_norm_w, W_mu, W_logvar, dec_norm_w, dec_Wg,
           dec_Wu, dec_Wd, dyn_norm_w, dyn_Wg, dyn_Wu, dyn_Wd, w_cr, b_cr):
    B, T, D = hidden_states.shape
    N = B * T
    BLK = 256
    nblk = N // BLK
    k = max(1, int(T * 0.25))

    x2d = hidden_states.reshape(N, D)
    row = lambda w: w.reshape(1, D)
    wcol = w_cr.reshape(D, 1)
    bcr = jnp.full((1, BLK), b_cr, jnp.float32)

    full = lambda shape: pl.BlockSpec(shape, lambda *_: (0,) * len(shape))
    tokb = pl.BlockSpec((BLK, D), lambda i: (i, 0))
    tpb = T // BLK

    proc, g2, flat_idx, gsel, loss = pl.pallas_call(
        functools.partial(_k12_body, k=k, n_tok=N, nblk=nblk, tpb=tpb),
        grid=(nblk,),
        in_specs=[
            tokb, full((1, D)), full((D, D_FF)), full((D, D_FF)),
            full((D_FF, D)), full((1, D)), full((D, D)), full((D, D)),
            full((D, 1)), full((1, BLK)),
        ],
        out_specs=[
            tokb,
            pl.BlockSpec((1, 1, BLK), lambda i: (i, 0, 0)),
            full((B, k)), full((B, k)), full((1, 128)),
        ],
        out_shape=[
            jax.ShapeDtypeStruct((N, D), jnp.float32),
            jax.ShapeDtypeStruct((nblk, 1, BLK), jnp.float32),
            jax.ShapeDtypeStruct((B, k), jnp.int32),
            jax.ShapeDtypeStruct((B, k), jnp.float32),
            jax.ShapeDtypeStruct((1, 128), jnp.float32),
        ],
        scratch_shapes=[
            pltpu.VMEM((B, T), jnp.float32),
            pltpu.VMEM((B, T), jnp.float32),
            pltpu.VMEM((B, T), jnp.float32),
        ],
        interpret=INTERPRET,
    )(x2d, row(dec_norm_w), dec_Wg, dec_Wu, dec_Wd, row(prior_norm_w), W_mu,
      W_logvar, wcol, bcr)

    g_bt = g2.reshape(B, T)
    flat_idx = flat_idx.reshape(B * k)

    mesh = plsc.VectorSubcoreMesh(core_axis_name="c", subcore_axis_name="s")
    gather_fn = functools.partial(
        pl.kernel,
        out_type=jax.ShapeDtypeStruct((B * k, D), jnp.float32),
        mesh=mesh,
        scratch_types=[
            pltpu.VMEM((k // 16,), jnp.int32),
            pltpu.VMEM((k // 16, D), jnp.float32),
            pltpu.SemaphoreType.DMA,
        ],
    )(functools.partial(_gather_body, K=k))
    sel = gather_fn(proc, flat_idx)

    gated = pl.pallas_call(
        _k4_body,
        grid=(B * k // BLK,),
        in_specs=[
            tokb, pl.BlockSpec((1, 1, BLK), lambda i: (i, 0, 0)), full((1, D)),
            full((D, D_FF)), full((D, D_FF)), full((D_FF, D)),
        ],
        out_specs=tokb,
        out_shape=jax.ShapeDtypeStruct((B * k, D), jnp.float32),
        interpret=INTERPRET,
    )(sel, gsel.reshape(B * k // BLK, 1, BLK), row(dyn_norm_w), dyn_Wg,
      dyn_Wu, dyn_Wd)

    scat_fn = functools.partial(
        pl.kernel,
        out_type=jax.ShapeDtypeStruct((N, D), jnp.float32),
        mesh=mesh,
        scratch_types=[
            pltpu.VMEM((32, D), jnp.float32),
            pltpu.VMEM((k // 16,), jnp.int32),
            pltpu.VMEM((k // 16, D), jnp.float32),
            pltpu.SemaphoreType.DMA,
        ],
    )(functools.partial(_scatter_body, T=T, K=k))
    final2d = scat_fn(proc, gated, flat_idx)

    return (final2d.reshape(B, T, D), g_bt, loss[0, 0])


# speedup vs baseline: 1.1497x; 1.0002x over previous
import functools

import jax
import jax.numpy as jnp
from jax import lax
from jax.experimental import pallas as pl
from jax.experimental.pallas import tpu as pltpu
from jax.experimental.pallas import tpu_sc as plsc

HIDDEN = 768
D_FF = 2048
EPS = 1e-06
INTERPRET = False


def _rms(x, w):
    v = jnp.mean(x * x, axis=-1, keepdims=True)
    return x * jax.lax.rsqrt(v + EPS) * w


def _k12_body(x_ref, decnw_ref, wg_ref, wu_ref, wd_ref, pnw_ref, wmu_ref,
              wlv_ref, wcr_ref, bcr_ref, proc_ref, g_ref, idx_ref, gsel_ref,
              loss_ref, pc_ref, mask_ref, blo_ref, g_s, logit_s, spl_s, *, k,
              n_tok, nblk, tpb):
    i = pl.program_id(0)
    x = x_ref[...]
    h = _rms(x, decnw_ref[...])
    hg = jnp.dot(h, wg_ref[...], preferred_element_type=jnp.float32)
    hu = jnp.dot(h, wu_ref[...], preferred_element_type=jnp.float32)
    act = (hg * jax.nn.sigmoid(hg)) * hu
    delta = jnp.dot(act, wd_ref[...], preferred_element_type=jnp.float32)
    proc_ref[...] = x + delta

    xn = _rms(x, pnw_ref[...])
    mu = jnp.dot(xn, wmu_ref[...], preferred_element_type=jnp.float32)
    lv = jnp.dot(xn, wlv_ref[...], preferred_element_type=jnp.float32)
    d_st = jnp.sum(delta * delta, axis=-1) / float(HIDDEN)
    d_ch = 0.5 * jnp.mean(
        lv + (1.0 + (delta - mu) ** 2) * jnp.exp(-lv) - 1.0, axis=-1)
    gv = jax.nn.sigmoid(d_st - d_ch)
    g_ref[0, 0, :] = gv

    logit = jnp.dot(x, wcr_ref[...],
                    preferred_element_type=jnp.float32)[:, 0] + bcr_ref[0, :]
    spl = jnp.maximum(logit, 0.0) + jnp.log1p(jnp.exp(-jnp.abs(logit)))

    b = i // tpb
    c0 = (i % tpb) * x.shape[0]
    sl = (pl.ds(b, 1), pl.ds(c0, x.shape[0]))
    g_s[sl] = gv.reshape(1, -1)
    logit_s[sl] = logit.reshape(1, -1)
    spl_s[sl] = spl.reshape(1, -1)

    @pl.when(i == nblk - 1)
    def _():
        _k2_work(g_s, logit_s, spl_s, idx_ref, gsel_ref, loss_ref, pc_ref,
                 mask_ref, blo_ref, k=k, n_tok=n_tok, blk=n_tok // nblk,
                 tpb=tpb)


def _k2_work(g_ref, logit_ref, s1_ref, idx_ref, gsel_ref, loss_ref, pc_ref,
             mask_ref, blo_ref, *, k, n_tok, blk, tpb):
    B, T = g_ref.shape
    g = g_ref[...]
    gi = jax.lax.bitcast_convert_type(g, jnp.int32)

    def bs_step(i, lo):
        cand = lo | jnp.left_shift(jnp.int32(1), 30 - i)
        cnt = jnp.sum((gi >= cand).astype(jnp.int32), axis=1, keepdims=True)
        return jnp.where(cnt >= k, cand, lo)

    lo = jnp.zeros((B, 1), jnp.int32)
    thr = jax.lax.fori_loop(0, 31, bs_step, lo)

    m_gt = gi > thr
    n_gt = jnp.sum(m_gt.astype(jnp.int32), axis=1, keepdims=True)
    r = k - n_gt
    m_eq = gi == thr
    c = m_eq.astype(jnp.int32)
    s = 1
    while s < T:
        c = c + jnp.concatenate(
            [jnp.zeros((B, s), jnp.int32), c[:, :-s]], axis=1)
        s *= 2
    mask = m_gt | (m_eq & (c <= r))

    lsel = jnp.sum(jnp.where(mask, logit_ref[...], 0.0))
    loss = (jnp.sum(s1_ref[...]) - lsel) / float(n_tok)
    loss_ref[0, :] = jnp.full((128,), loss)

    mi = mask.astype(jnp.int32)
    pc = mi
    s = 1
    while s < T:
        pc = pc + jnp.concatenate(
            [jnp.zeros((B, s), jnp.int32), pc[:, :-s]], axis=1)
        s *= 2

    PK = 128
    tio = lax.broadcasted_iota(jnp.int32, (1, T), 1)
    rows_idx = []
    rows_g = []
    for b in range(B):
        pcb = pc[b:b + 1, :]
        mb = mask[b:b + 1, :]
        gb = g[b:b + 1, :]
        chunks_i = []
        chunks_g = []
        for c0 in range(0, k, PK):
            prow = lax.broadcasted_iota(jnp.int32, (PK, 1), 0) + (c0 + 1)
            eqm = (pcb == prow) & mb
            chunks_i.append(
                jnp.sum(jnp.where(eqm, tio + b * T, 0), axis=1))
            chunks_g.append(jnp.sum(jnp.where(eqm, gb, 0.0), axis=1))
        rows_idx.append(jnp.concatenate(chunks_i))
        rows_g.append(jnp.concatenate(chunks_g))
    idx_ref[...] = jnp.stack(rows_idx)
    gsel_ref[...] = jnp.stack(rows_g)

    nblk = n_tok // blk
    pc_ref[...] = pc.reshape(nblk, 1, blk)
    mask_ref[...] = mask.astype(jnp.float32).reshape(nblk, 1, blk)
    los = []
    for j in range(nblk):
        b = j // tpb
        m = j % tpb
        if m == 0:
            los.append(jnp.full((), b * k, jnp.int32))
        else:
            v = jnp.sum(jnp.where(tio == m * blk - 1, pc[b:b + 1, :], 0))
            los.append((v + b * k).astype(jnp.int32))
    blo = jnp.stack(los)
    blo_ref[0, :] = jnp.concatenate(
        [blo, jnp.zeros((128 - nblk,), jnp.int32)])


def _gather_body(proc_hbm, idx_hbm, sel_hbm, idx_v, rows_v, sem, *, K):
    c = lax.axis_index("c")
    s = lax.axis_index("s")
    bpw = K // 16
    base = c * K + s * bpw
    pltpu.sync_copy(idx_hbm.at[pl.ds(base, bpw)], idx_v)
    pltpu.async_copy(proc_hbm.at[idx_v], rows_v, sem).wait()
    pltpu.sync_copy(rows_v, sel_hbm.at[pl.ds(base, bpw)])


def _k45_body(blo_ref, sel_ref, gsel_ref, nw_ref, wg_ref, wu_ref, wd_ref,
              proc_ref, pc_ref, mask_ref, out_ref, gated_s, *, k, blk, tpb,
              nph_a):
    i = pl.program_id(0)

    @pl.when(i < nph_a)
    def _():
        p = sel_ref[...]
        h = _rms(p, nw_ref[...])
        hg = jnp.dot(h, wg_ref[...], preferred_element_type=jnp.float32)
        hu = jnp.dot(h, wu_ref[...], preferred_element_type=jnp.float32)
        act = (hg * jax.nn.sigmoid(hg)) * hu
        delta = jnp.dot(act, wd_ref[...], preferred_element_type=jnp.float32)
        gated_s[pl.ds(i * blk, blk), :] = (
            p + gsel_ref[...].reshape(-1, 1) * delta)

    @pl.when(i >= nph_a)
    def _():
        j = i - nph_a
        b = j // tpb
        lo = blo_ref[j]
        lo_al = (lo // 8) * 8
        win_n = blk + 8
        psec = pc_ref[0, 0, :]
        mseg = mask_ref[0, 0, :]
        win = gated_s[pl.ds(lo_al, win_n), :]
        sl = b * k + psec - 1 - lo_al
        colio = lax.broadcasted_iota(jnp.int32, (blk, win_n), 1)
        p1 = jnp.where(
            (sl.reshape(blk, 1) == colio) & (mseg.reshape(blk, 1) > 0.0),
            1.0, 0.0)
        upd = jnp.dot(p1, win, preferred_element_type=jnp.float32)
        out_ref[...] = proc_ref[...] * (1.0 - mseg.reshape(blk, 1)) + upd


def kernel(hidden_states, prior_norm_w, W_mu, W_logvar, dec_norm_w, dec_Wg,
           dec_Wu, dec_Wd, dyn_norm_w, dyn_Wg, dyn_Wu, dyn_Wd, w_cr, b_cr):
    B, T, D = hidden_states.shape
    N = B * T
    BLK = 256
    nblk = N // BLK
    k = max(1, int(T * 0.25))

    x2d = hidden_states.reshape(N, D)
    row = lambda w: w.reshape(1, D)
    wcol = w_cr.reshape(D, 1)
    bcr = jnp.full((1, BLK), b_cr, jnp.float32)

    full = lambda shape: pl.BlockSpec(shape, lambda *_: (0,) * len(shape))
    tokb = pl.BlockSpec((BLK, D), lambda i: (i, 0))
    tpb = T // BLK

    proc, g2, flat_idx, gsel, loss, pc3, mask3, blo = pl.pallas_call(
        functools.partial(_k12_body, k=k, n_tok=N, nblk=nblk, tpb=tpb),
        grid=(nblk,),
        in_specs=[
            tokb, full((1, D)), full((D, D_FF)), full((D, D_FF)),
            full((D_FF, D)), full((1, D)), full((D, D)), full((D, D)),
            full((D, 1)), full((1, BLK)),
        ],
        out_specs=[
            tokb,
            pl.BlockSpec((1, 1, BLK), lambda i: (i, 0, 0)),
            full((B, k)), full((B, k)), full((1, 128)),
            full((nblk, 1, BLK)), full((nblk, 1, BLK)), full((1, 128)),
        ],
        out_shape=[
            jax.ShapeDtypeStruct((N, D), jnp.float32),
            jax.ShapeDtypeStruct((nblk, 1, BLK), jnp.float32),
            jax.ShapeDtypeStruct((B, k), jnp.int32),
            jax.ShapeDtypeStruct((B, k), jnp.float32),
            jax.ShapeDtypeStruct((1, 128), jnp.float32),
            jax.ShapeDtypeStruct((nblk, 1, BLK), jnp.int32),
            jax.ShapeDtypeStruct((nblk, 1, BLK), jnp.float32),
            jax.ShapeDtypeStruct((1, 128), jnp.int32),
        ],
        scratch_shapes=[
            pltpu.VMEM((B, T), jnp.float32),
            pltpu.VMEM((B, T), jnp.float32),
            pltpu.VMEM((B, T), jnp.float32),
        ],
        interpret=INTERPRET,
    )(x2d, row(dec_norm_w), dec_Wg, dec_Wu, dec_Wd, row(prior_norm_w), W_mu,
      W_logvar, wcol, bcr)

    g_bt = g2.reshape(B, T)
    flat_idx = flat_idx.reshape(B * k)

    mesh = plsc.VectorSubcoreMesh(core_axis_name="c", subcore_axis_name="s")
    gather_fn = functools.partial(
        pl.kernel,
        out_type=jax.ShapeDtypeStruct((B * k, D), jnp.float32),
        mesh=mesh,
        scratch_types=[
            pltpu.VMEM((k // 16,), jnp.int32),
            pltpu.VMEM((k // 16, D), jnp.float32),
            pltpu.SemaphoreType.DMA,
        ],
    )(functools.partial(_gather_body, K=k))
    sel = gather_fn(proc, flat_idx)

    nph_a = B * k // BLK
    final2d = pl.pallas_call(
        functools.partial(_k45_body, k=k, blk=BLK, tpb=tpb, nph_a=nph_a),
        grid=(nph_a + nblk,),
        in_specs=[
            pl.BlockSpec(memory_space=pltpu.SMEM),
            pl.BlockSpec((BLK, D), lambda i: (jnp.minimum(i, nph_a - 1), 0)),
            pl.BlockSpec((1, 1, BLK),
                         lambda i: (jnp.minimum(i, nph_a - 1), 0, 0)),
            full((1, D)), full((D, D_FF)), full((D, D_FF)), full((D_FF, D)),
            pl.BlockSpec((BLK, D), lambda i: (jnp.maximum(i - nph_a, 0), 0)),
            pl.BlockSpec((1, 1, BLK),
                         lambda i: (jnp.maximum(i - nph_a, 0), 0, 0)),
            pl.BlockSpec((1, 1, BLK),
                         lambda i: (jnp.maximum(i - nph_a, 0), 0, 0)),
        ],
        out_specs=pl.BlockSpec((BLK, D),
                               lambda i: (jnp.maximum(i - nph_a, 0), 0)),
        out_shape=jax.ShapeDtypeStruct((N, D), jnp.float32),
        scratch_shapes=[pltpu.VMEM((B * k + BLK + 8, D), jnp.float32)],
        interpret=INTERPRET,
    )(blo.reshape(128), sel, gsel.reshape(nph_a, 1, BLK), row(dyn_norm_w),
      dyn_Wg, dyn_Wu, dyn_Wd, proc, pc3, mask3)

    return (final2d.reshape(B, T, D), g_bt, loss[0, 0])
